# Initial kernel scaffold; baseline (speedup 1.0000x reference)
import jax
import jax.numpy as jnp
from jax.experimental import pallas as pl


def kernel(x, edge_features, edge_idx, batch_idx, M_W, M_b, U_W, U_b):
    raise NotImplementedError("write your pallas kernel here")


# speedup vs baseline: 2.2622x; 2.2622x over previous
import functools

import jax
import jax.numpy as jnp
from jax import lax
from jax.experimental import pallas as pl
from jax.experimental.pallas import tpu as pltpu
from jax.experimental.pallas import tpu_sc as plsc

_NEG = -1e9
_NC = 2
_NS = 16
_LANES = 16
_IDXW = 128
_CHUNK_ROWS = 4


def _em_body(ef2_ref, w_ref, b_ref, o_ref, *, n_edges, blk):
    pid = pl.program_id(0)
    acc = jnp.dot(ef2_ref[...], w_ref[...],
                  preferred_element_type=jnp.float32) + b_ref[...]
    rows = pid * blk + lax.broadcasted_iota(jnp.int32, (blk, 128), 0)
    lane = lax.broadcasted_iota(jnp.int32, (blk, 128), 1)
    edge_id = 2 * rows + lane // 64
    o_ref[...] = jnp.where(edge_id < n_edges, acc, _NEG)


def _xm_body(x2_ref, w_ref, o_ref):
    o_ref[...] = jnp.dot(x2_ref[...], w_ref[...],
                         preferred_element_type=jnp.float32)


def _out_body(x_ref, p0_ref, p1_ref, wx_ref, wa_ref, b_ref, o_ref):
    agg = p0_ref[...] + p1_ref[...]
    o_ref[...] = (jnp.dot(x_ref[...], wx_ref[...],
                          preferred_element_type=jnp.float32)
                  + jnp.dot(agg, wa_ref[...],
                            preferred_element_type=jnp.float32)
                  + b_ref[...])


def _sc_body(xm_hbm, em_hbm, src_hbm, dst_hbm, out_hbm,
             idx_s, idx_d, gath, emb, agg_sh, sem,
             *, n_nodes, dim_m, rows_per_worker):
    cid = lax.axis_index("c")
    sid = lax.axis_index("s")
    wid = sid * _NC + cid
    cpe = _CHUNK_ROWS * _IDXW
    nodes_per_tile = n_nodes // _NS
    vregs_per_row = dim_m // _LANES

    def zrow(r, c):
        for cc in range(vregs_per_row):
            gath[r, pl.ds(cc * _LANES, _LANES)] = jnp.zeros((_LANES,), jnp.float32)
        return c
    lax.fori_loop(0, cpe, zrow, 0)
    node_base = sid * nodes_per_tile
    full = min(cpe, nodes_per_tile)
    pltpu.sync_copy(gath.at[pl.ds(0, full)], agg_sh.at[pl.ds(node_base, full)])
    rem = nodes_per_tile - full
    if rem > 0:
        pltpu.sync_copy(gath.at[pl.ds(0, rem)],
                        agg_sh.at[pl.ds(node_base + full, rem)])
    plsc.subcore_barrier()

    def chunk(i, c):
        base_row = wid * rows_per_worker + i * _CHUNK_ROWS
        pltpu.sync_copy(src_hbm.at[pl.ds(base_row, _CHUNK_ROWS)], idx_s)
        pltpu.sync_copy(dst_hbm.at[pl.ds(base_row, _CHUNK_ROWS)], idx_d)
        pltpu.sync_copy(em_hbm.at[pl.ds(base_row * (_IDXW // 2), cpe // 2)], emb)
        cps = [pltpu.async_copy(xm_hbm.at[idx_s.at[j]],
                                gath.at[pl.ds(j * _IDXW, _IDXW)], sem)
               for j in range(_CHUNK_ROWS)]
        for cp in cps:
            cp.wait()

        def comp(re, cc_):
            for half in range(2):
                r = 2 * re + half
                for cc in range(vregs_per_row):
                    s = pl.ds(cc * _LANES, _LANES)
                    se = pl.ds(half * dim_m + cc * _LANES, _LANES)
                    gath[r, s] = jnp.maximum(gath[r, s] + emb[re, se], 0.0)
            return cc_
        lax.fori_loop(0, cpe // 2, comp, 0)

        for j in range(_CHUNK_ROWS):
            pltpu.sync_copy(gath.at[pl.ds(j * _IDXW, _IDXW)],
                            agg_sh.at[idx_d.at[j]], add=True)
        return c
    lax.fori_loop(0, rows_per_worker // _CHUNK_ROWS, chunk, 0)

    plsc.subcore_barrier()
    pltpu.sync_copy(agg_sh.at[pl.ds(node_base, nodes_per_tile)],
                    out_hbm.at[cid * _NS + sid])


def kernel(x, edge_features, edge_idx, batch_idx, M_W, M_b, U_W, U_b):
    n_nodes, dim_in = x.shape
    n_edges, dim_e = edge_features.shape
    dim_m = M_W.shape[0]
    dim_out = U_W.shape[0]

    n_workers = _NC * _NS
    cpe = _CHUNK_ROWS * _IDXW
    e_pad = ((n_edges + n_workers * cpe - 1) // (n_workers * cpe)) * (n_workers * cpe)
    rows_per_worker = e_pad // n_workers // _IDXW

    src = jnp.pad(edge_idx[0], (0, e_pad - n_edges)).reshape(e_pad // _IDXW, _IDXW)
    dst = jnp.pad(edge_idx[1], (0, e_pad - n_edges)).reshape(e_pad // _IDXW, _IDXW)

    pair = 128 // dim_m
    w_e = M_W[:, dim_in:]
    w2e = jnp.zeros((pair * dim_e, pair * dim_m), jnp.float32)
    for s in range(pair):
        w2e = w2e.at[s * dim_e:(s + 1) * dim_e,
                     s * dim_m:(s + 1) * dim_m].set(w_e.T)
    b2 = jnp.tile(M_b, (pair,)).reshape(1, pair * dim_m)
    w_x = M_W[:, :dim_in]
    w2x = jnp.zeros((pair * dim_in, pair * dim_m), jnp.float32)
    for s in range(pair):
        w2x = w2x.at[s * dim_in:(s + 1) * dim_in,
                     s * dim_m:(s + 1) * dim_m].set(w_x.T)

    ef2 = jnp.pad(edge_features,
                  ((0, e_pad - n_edges), (0, 0))).reshape(e_pad // pair,
                                                          pair * dim_e)

    blk_e = 4096
    em = pl.pallas_call(
        functools.partial(_em_body, n_edges=n_edges, blk=blk_e),
        grid=(e_pad // pair // blk_e,),
        in_specs=[pl.BlockSpec((blk_e, pair * dim_e), lambda i: (i, 0)),
                  pl.BlockSpec((pair * dim_e, pair * dim_m), lambda i: (0, 0)),
                  pl.BlockSpec((1, pair * dim_m), lambda i: (0, 0))],
        out_specs=pl.BlockSpec((blk_e, pair * dim_m), lambda i: (i, 0)),
        out_shape=jax.ShapeDtypeStruct((e_pad // pair, pair * dim_m),
                                       jnp.float32),
    )(ef2, w2e, b2)

    blk_n = 2000
    x2 = x.reshape(n_nodes // pair, pair * dim_in)
    xm2 = pl.pallas_call(
        _xm_body,
        grid=(n_nodes // pair // (blk_n // pair),),
        in_specs=[pl.BlockSpec((blk_n // pair, pair * dim_in), lambda i: (i, 0)),
                  pl.BlockSpec((pair * dim_in, pair * dim_m), lambda i: (0, 0))],
        out_specs=pl.BlockSpec((blk_n // pair, pair * dim_m), lambda i: (i, 0)),
        out_shape=jax.ShapeDtypeStruct((n_nodes // pair, pair * dim_m),
                                       jnp.float32),
    )(x2, w2x)
    xm = xm2.reshape(n_nodes, dim_m)

    mesh = plsc.VectorSubcoreMesh(core_axis_name="c", subcore_axis_name="s",
                                  num_cores=_NC, num_subcores=_NS)
    sc = pl.kernel(
        functools.partial(_sc_body, n_nodes=n_nodes, dim_m=dim_m,
                          rows_per_worker=rows_per_worker),
        out_type=jax.ShapeDtypeStruct((_NC * _NS, n_nodes // _NS, dim_m),
                                      jnp.float32),
        mesh=mesh,
        compiler_params=pltpu.CompilerParams(use_tc_tiling_on_sc=False),
        scratch_types=[
            pltpu.VMEM((_CHUNK_ROWS, _IDXW), jnp.int32),
            pltpu.VMEM((_CHUNK_ROWS, _IDXW), jnp.int32),
            pltpu.VMEM((cpe, dim_m), jnp.float32),
            pltpu.VMEM((cpe // 2, pair * dim_m), jnp.float32),
            pltpu.VMEM_SHARED((n_nodes, dim_m), jnp.float32),
            pltpu.SemaphoreType.DMA,
        ],
    )
    partials = sc(xm, em, src, dst).reshape(_NC, n_nodes, dim_m)

    out = pl.pallas_call(
        _out_body,
        grid=(n_nodes // blk_n,),
        in_specs=[pl.BlockSpec((blk_n, dim_in), lambda i: (i, 0)),
                  pl.BlockSpec((blk_n, dim_m), lambda i: (i, 0)),
                  pl.BlockSpec((blk_n, dim_m), lambda i: (i, 0)),
                  pl.BlockSpec((dim_in, dim_out), lambda i: (0, 0)),
                  pl.BlockSpec((dim_m, dim_out), lambda i: (0, 0)),
                  pl.BlockSpec((1, dim_out), lambda i: (0, 0))],
        out_specs=pl.BlockSpec((blk_n, dim_out), lambda i: (i, 0)),
        out_shape=jax.ShapeDtypeStruct((n_nodes, dim_out), jnp.float32),
    )(x, partials[0], partials[1],
      U_W[:, :dim_in].T, U_W[:, dim_in:].T, U_b.reshape(1, dim_out))
    return out


# speedup vs baseline: 3.3947x; 1.5006x over previous
import functools

import jax
import jax.numpy as jnp
from jax import lax
from jax.experimental import pallas as pl
from jax.experimental.pallas import tpu as pltpu
from jax.experimental.pallas import tpu_sc as plsc

_NEG = -1e9
_NC = 2
_NS = 16
_LANES = 16
_IDXW = 128
_CROWS = 1
_CPE = _CROWS * _IDXW
_RING = 4
_NCHUNK = 80


def _em_body(ef2_ref, w_ref, b_ref, o_ref, *, n_edges, blk):
    pid = pl.program_id(0)
    acc = jnp.dot(ef2_ref[...], w_ref[...],
                  preferred_element_type=jnp.float32) + b_ref[...]
    rows = pid * blk + lax.broadcasted_iota(jnp.int32, (blk, 128), 0)
    lane = lax.broadcasted_iota(jnp.int32, (blk, 128), 1)
    edge_id = 2 * rows + lane // 64
    o_ref[...] = jnp.where(edge_id < n_edges, acc, _NEG)


def _xm_body(x2_ref, w_ref, o_ref):
    o_ref[...] = jnp.dot(x2_ref[...], w_ref[...],
                         preferred_element_type=jnp.float32)


def _out_body(x_ref, p0_ref, p1_ref, wx_ref, wa_ref, b_ref, o_ref):
    agg = p0_ref[...] + p1_ref[...]
    o_ref[...] = (jnp.dot(x_ref[...], wx_ref[...],
                          preferred_element_type=jnp.float32)
                  + jnp.dot(agg, wa_ref[...],
                            preferred_element_type=jnp.float32)
                  + b_ref[...])


def _sc_body(xm_hbm, em_hbm, idx_hbm, out_hbm,
             idx, gath, emb, agg_sh,
             semg0, semg1, sems0, sems1, sems2, sems3,
             *, n_nodes, dim_m, rows_per_worker):
    cid = lax.axis_index("c")
    sid = lax.axis_index("s")
    wid = sid * _NC + cid
    nodes_per_tile = n_nodes // _NS
    vregs = dim_m // _LANES
    emrows = _CPE // 2
    semg = [semg0, semg1]
    sems = [sems0, sems1, sems2, sems3]
    base0 = wid * rows_per_worker

    def gather_fire(c, b):
        for j in range(_CROWS):
            pltpu.async_copy(xm_hbm.at[idx.at[b % _RING, j, 0]],
                             gath.at[b % _RING, pl.ds(j * _IDXW, _IDXW)],
                             semg[b % 2])
        pltpu.async_copy(em_hbm.at[pl.ds((base0 + c * _CROWS) * (_IDXW // 2),
                                         emrows)],
                         emb.at[b % 2], semg[b % 2])

    def gather_wait(b):
        for j in range(_CROWS):
            pltpu.make_async_copy(
                xm_hbm.at[pl.ds(0, _IDXW)],
                gath.at[b % _RING, pl.ds(j * _IDXW, _IDXW)],
                semg[b % 2]).wait()
        pltpu.make_async_copy(em_hbm.at[pl.ds(0, emrows)],
                              emb.at[b % 2], semg[b % 2]).wait()

    def idx_load(c, b):
        pltpu.sync_copy(idx_hbm.at[pl.ds(base0 + c * _CROWS, _CROWS)],
                        idx.at[b % _RING])

    def scatter_fire(b):
        for j in range(_CROWS):
            pltpu.sync_copy(gath.at[b % _RING, pl.ds(j * _IDXW, _IDXW)],
                            agg_sh.at[idx.at[b % _RING, j, 1]], add=True)

    def scatter_wait(b):
        del b

    def zrow(r, c):
        for cc in range(vregs):
            gath[0, r, pl.ds(cc * _LANES, _LANES)] = jnp.zeros((_LANES,),
                                                               jnp.float32)
        return c
    lax.fori_loop(0, _CPE, zrow, 0)
    node_base = sid * nodes_per_tile
    done = 0
    while done < nodes_per_tile:
        n = min(_CPE, nodes_per_tile - done)
        pltpu.sync_copy(gath.at[0, pl.ds(0, n)],
                        agg_sh.at[pl.ds(node_base + done, n)])
        done += n
    plsc.subcore_barrier()

    for k in range(2):
        idx_load(k, k)
        gather_fire(k, k)

    def outer(o, carry):
        for b in range(_RING):
            i = o * _RING + b
            gather_wait(b)

            def comp(re, cc_):
                for half in range(2):
                    for cc in range(vregs):
                        s = pl.ds(cc * _LANES, _LANES)
                        se = pl.ds(half * dim_m + cc * _LANES, _LANES)
                        v = gath[b, 2 * re + half, s] + emb[b % 2, re, se]
                        gath[b, 2 * re + half, s] = jnp.maximum(v, 0.0)
                return cc_
            lax.fori_loop(0, emrows, comp, 0)

            scatter_fire(b)

            @pl.when(i + 2 < _NCHUNK)
            def _prefetch():
                @pl.when(i >= 2)
                def _drain():
                    scatter_wait(b + 2)
                idx_load(i + 2, b + 2)
                gather_fire(i + 2, b + 2)
        return carry
    lax.fori_loop(0, _NCHUNK // _RING, outer, 0)

    for b in range(_RING):
        scatter_wait(b)

    plsc.subcore_barrier()
    pltpu.sync_copy(agg_sh.at[pl.ds(node_base, nodes_per_tile)],
                    out_hbm.at[cid * _NS + sid])


def kernel(x, edge_features, edge_idx, batch_idx, M_W, M_b, U_W, U_b):
    n_nodes, dim_in = x.shape
    n_edges, dim_e = edge_features.shape
    dim_m = M_W.shape[0]
    dim_out = U_W.shape[0]

    n_workers = _NC * _NS
    e_pad = n_workers * _NCHUNK * _CPE
    rows_per_worker = e_pad // n_workers // _IDXW

    n_pad = e_pad - n_edges
    pad_idx = (jnp.arange(n_pad, dtype=jnp.int32) * 7) % n_nodes
    src = jnp.concatenate([edge_idx[0], pad_idx]).reshape(e_pad // _IDXW, _IDXW)
    dst = jnp.concatenate([edge_idx[1], pad_idx]).reshape(e_pad // _IDXW, _IDXW)
    comb = jnp.stack([src, dst], axis=1)

    pair = 128 // dim_m
    w_e = M_W[:, dim_in:]
    w2e = jnp.zeros((pair * dim_e, pair * dim_m), jnp.float32)
    for s in range(pair):
        w2e = w2e.at[s * dim_e:(s + 1) * dim_e,
                     s * dim_m:(s + 1) * dim_m].set(w_e.T)
    b2 = jnp.tile(M_b, (pair,)).reshape(1, pair * dim_m)
    w_x = M_W[:, :dim_in]
    w2x = jnp.zeros((pair * dim_in, pair * dim_m), jnp.float32)
    for s in range(pair):
        w2x = w2x.at[s * dim_in:(s + 1) * dim_in,
                     s * dim_m:(s + 1) * dim_m].set(w_x.T)

    ef2 = jnp.pad(edge_features,
                  ((0, n_pad), (0, 0))).reshape(e_pad // pair, pair * dim_e)

    blk_e = 4096
    em = pl.pallas_call(
        functools.partial(_em_body, n_edges=n_edges, blk=blk_e),
        grid=(e_pad // pair // blk_e,),
        in_specs=[pl.BlockSpec((blk_e, pair * dim_e), lambda i: (i, 0)),
                  pl.BlockSpec((pair * dim_e, pair * dim_m), lambda i: (0, 0)),
                  pl.BlockSpec((1, pair * dim_m), lambda i: (0, 0))],
        out_specs=pl.BlockSpec((blk_e, pair * dim_m), lambda i: (i, 0)),
        out_shape=jax.ShapeDtypeStruct((e_pad // pair, pair * dim_m),
                                       jnp.float32),
    )(ef2, w2e, b2)

    blk_n = 2000
    x2 = x.reshape(n_nodes // pair, pair * dim_in)
    xm2 = pl.pallas_call(
        _xm_body,
        grid=(n_nodes // pair // (blk_n // pair),),
        in_specs=[pl.BlockSpec((blk_n // pair, pair * dim_in), lambda i: (i, 0)),
                  pl.BlockSpec((pair * dim_in, pair * dim_m), lambda i: (0, 0))],
        out_specs=pl.BlockSpec((blk_n // pair, pair * dim_m), lambda i: (i, 0)),
        out_shape=jax.ShapeDtypeStruct((n_nodes // pair, pair * dim_m),
                                       jnp.float32),
    )(x2, w2x)
    xm = xm2.reshape(n_nodes, dim_m)

    mesh = plsc.VectorSubcoreMesh(core_axis_name="c", subcore_axis_name="s",
                                  num_cores=_NC, num_subcores=_NS)
    sc = pl.kernel(
        functools.partial(_sc_body, n_nodes=n_nodes, dim_m=dim_m,
                          rows_per_worker=rows_per_worker),
        out_type=jax.ShapeDtypeStruct((_NC * _NS, n_nodes // _NS, dim_m),
                                      jnp.float32),
        mesh=mesh,
        compiler_params=pltpu.CompilerParams(use_tc_tiling_on_sc=False),
        scratch_types=[
            pltpu.VMEM((_RING, _CROWS, 2, _IDXW), jnp.int32),
            pltpu.VMEM((_RING, _CPE, dim_m), jnp.float32),
            pltpu.VMEM((2, _CPE // 2, pair * dim_m), jnp.float32),
            pltpu.VMEM_SHARED((n_nodes, dim_m), jnp.float32),
            pltpu.SemaphoreType.DMA,
            pltpu.SemaphoreType.DMA,
            pltpu.SemaphoreType.DMA,
            pltpu.SemaphoreType.DMA,
            pltpu.SemaphoreType.DMA,
            pltpu.SemaphoreType.DMA,
        ],
    )
    partials = sc(xm, em, comb).reshape(_NC, n_nodes, dim_m)

    out = pl.pallas_call(
        _out_body,
        grid=(n_nodes // blk_n,),
        in_specs=[pl.BlockSpec((blk_n, dim_in), lambda i: (i, 0)),
                  pl.BlockSpec((blk_n, dim_m), lambda i: (i, 0)),
                  pl.BlockSpec((blk_n, dim_m), lambda i: (i, 0)),
                  pl.BlockSpec((dim_in, dim_out), lambda i: (0, 0)),
                  pl.BlockSpec((dim_m, dim_out), lambda i: (0, 0)),
                  pl.BlockSpec((1, dim_out), lambda i: (0, 0))],
        out_specs=pl.BlockSpec((blk_n, dim_out), lambda i: (i, 0)),
        out_shape=jax.ShapeDtypeStruct((n_nodes, dim_out), jnp.float32),
    )(x, partials[0], partials[1],
      U_W[:, :dim_in].T, U_W[:, dim_in:].T, U_b.reshape(1, dim_out))
    return out


# speedup vs baseline: 3.5265x; 1.0388x over previous
import functools

import jax
import jax.numpy as jnp
from jax import lax
from jax.experimental import pallas as pl
from jax.experimental.pallas import tpu as pltpu
from jax.experimental.pallas import tpu_sc as plsc

_NC = 2
_NS = 16
_LANES = 16
_CPE = 80
_RING = 5
_NCHUNK = 125


def _em_body(efa_ref, efb_ref, w_ref, b_ref, o_ref):
    o_ref[:, 0:64] = jnp.dot(efa_ref[...], w_ref[...],
                             preferred_element_type=jnp.float32) + b_ref[...]
    o_ref[:, 64:128] = jnp.dot(efb_ref[...], w_ref[...],
                               preferred_element_type=jnp.float32) + b_ref[...]


def _xm_body(x2_ref, w_ref, o_ref):
    o_ref[...] = jnp.dot(x2_ref[...], w_ref[...],
                         preferred_element_type=jnp.float32)


def _out_body(x_ref, p0_ref, p1_ref, wx_ref, wa_ref, b_ref, o_ref):
    agg = p0_ref[...] + p1_ref[...]
    o_ref[...] = (jnp.dot(x_ref[...], wx_ref[...],
                          preferred_element_type=jnp.float32)
                  + jnp.dot(agg, wa_ref[...],
                            preferred_element_type=jnp.float32)
                  + b_ref[...])


def _sc_body(xm_hbm, em_hbm, idx_hbm, out_hbm,
             idx, gath, emb, agg_sh,
             semg0, semg1, semg2, semg3, semg4, sems0, sems1,
             *, n_nodes, dim_m):
    cid = lax.axis_index("c")
    sid = lax.axis_index("s")
    wid = sid * _NC + cid
    nodes_per_tile = n_nodes // _NS
    vregs = dim_m // _LANES
    half = _CPE // 2
    semg = [semg0, semg1, semg2, semg3, semg4]
    sems = [sems0, sems1]
    g0 = wid * _NCHUNK

    def gather_fire(c, b):
        pltpu.async_copy(xm_hbm.at[idx.at[b % _RING, 0]],
                         gath.at[b % _RING], semg[b % _RING])
        pltpu.async_copy(em_hbm.at[pl.ds((g0 + c) * half, half)],
                         emb.at[b % _RING], semg[b % _RING])

    def gather_wait(b):
        pltpu.make_async_copy(xm_hbm.at[pl.ds(0, _CPE)],
                              gath.at[b % _RING], semg[b % _RING]).wait()
        pltpu.make_async_copy(em_hbm.at[pl.ds(0, half)],
                              emb.at[b % _RING], semg[b % _RING]).wait()

    def idx_load(c, b):
        pltpu.sync_copy(idx_hbm.at[pl.ds(2 * (g0 + c), 2)],
                        idx.at[b % _RING])

    def scatter_fire(b):
        return pltpu.async_copy(gath.at[b % _RING],
                                agg_sh.at[idx.at[b % _RING, 1]],
                                sems[b % 2], add=True)

    def zrow(r, c):
        for cc in range(vregs):
            gath[0, r, pl.ds(cc * _LANES, _LANES)] = jnp.zeros((_LANES,),
                                                               jnp.float32)
        return c
    lax.fori_loop(0, _CPE, zrow, 0)
    node_base = sid * nodes_per_tile
    done = 0
    while done < nodes_per_tile:
        n = min(_CPE, nodes_per_tile - done)
        pltpu.sync_copy(gath.at[0, pl.ds(0, n)],
                        agg_sh.at[pl.ds(node_base + done, n)])
        done += n
    plsc.subcore_barrier()

    for k in range(2):
        idx_load(k, k)
        gather_fire(k, k)

    def outer(o, carry):
        descs = [None] * _RING
        for b in range(_RING):
            i = o * _RING + b
            gather_wait(b)

            def comp(j, cc_, b=b):
                for h in range(2):
                    for cc in range(vregs):
                        s = pl.ds(cc * _LANES, _LANES)
                        se = pl.ds(h * dim_m + cc * _LANES, _LANES)
                        v = gath[b, h * half + j, s] + emb[b, j, se]
                        gath[b, h * half + j, s] = jnp.maximum(v, 0.0)
                return cc_
            lax.fori_loop(0, half, comp, 0)

            descs[b] = scatter_fire(b)
            if b >= 1:
                descs[b - 1].wait()

            @pl.when(i + 2 < _NCHUNK)
            def _prefetch(i=i, b=b):
                idx_load(i + 2, b + 2)
                gather_fire(i + 2, b + 2)
        descs[_RING - 1].wait()
        return carry
    lax.fori_loop(0, _NCHUNK // _RING, outer, 0)

    plsc.subcore_barrier()
    pltpu.sync_copy(agg_sh.at[pl.ds(node_base, nodes_per_tile)],
                    out_hbm.at[cid * _NS + sid])


def kernel(x, edge_features, edge_idx, batch_idx, M_W, M_b, U_W, U_b):
    n_nodes, dim_in = x.shape
    n_edges, dim_e = edge_features.shape
    dim_m = M_W.shape[0]
    dim_out = U_W.shape[0]

    n_workers = _NC * _NS
    assert n_edges == n_workers * _NCHUNK * _CPE
    e_half = n_edges // 2
    half = _CPE // 2
    n_rows = e_half // half

    def chunk_rows(v):
        lo = v[:e_half].reshape(n_rows, half)
        hi = v[e_half:].reshape(n_rows, half)
        return jnp.concatenate([lo, hi], axis=1)
    comb = jnp.stack([chunk_rows(edge_idx[0]), chunk_rows(edge_idx[1])],
                     axis=1).reshape(2 * n_rows, _CPE)

    blk_e = 4000
    grid_e = e_half // blk_e
    em = pl.pallas_call(
        _em_body,
        grid=(grid_e,),
        in_specs=[pl.BlockSpec((blk_e, dim_e), lambda i: (i, 0)),
                  pl.BlockSpec((blk_e, dim_e), lambda i, g=grid_e: (i + g, 0)),
                  pl.BlockSpec((dim_e, dim_m), lambda i: (0, 0)),
                  pl.BlockSpec((1, dim_m), lambda i: (0, 0))],
        out_specs=pl.BlockSpec((blk_e, 2 * dim_m), lambda i: (i, 0)),
        out_shape=jax.ShapeDtypeStruct((e_half, 2 * dim_m), jnp.float32),
    )(edge_features, edge_features, M_W[:, dim_in:].T, M_b.reshape(1, dim_m))

    pair = 128 // dim_m
    w_x = M_W[:, :dim_in]
    w2x = jnp.zeros((pair * dim_in, pair * dim_m), jnp.float32)
    for s in range(pair):
        w2x = w2x.at[s * dim_in:(s + 1) * dim_in,
                     s * dim_m:(s + 1) * dim_m].set(w_x.T)
    blk_n = 2000
    x2 = x.reshape(n_nodes // pair, pair * dim_in)
    xm2 = pl.pallas_call(
        _xm_body,
        grid=(n_nodes // pair // (blk_n // pair),),
        in_specs=[pl.BlockSpec((blk_n // pair, pair * dim_in), lambda i: (i, 0)),
                  pl.BlockSpec((pair * dim_in, pair * dim_m), lambda i: (0, 0))],
        out_specs=pl.BlockSpec((blk_n // pair, pair * dim_m), lambda i: (i, 0)),
        out_shape=jax.ShapeDtypeStruct((n_nodes // pair, pair * dim_m),
                                       jnp.float32),
    )(x2, w2x)
    xm = xm2.reshape(n_nodes, dim_m)

    mesh = plsc.VectorSubcoreMesh(core_axis_name="c", subcore_axis_name="s",
                                  num_cores=_NC, num_subcores=_NS)
    sc = pl.kernel(
        functools.partial(_sc_body, n_nodes=n_nodes, dim_m=dim_m),
        out_type=jax.ShapeDtypeStruct((_NC * _NS, n_nodes // _NS, dim_m),
                                      jnp.float32),
        mesh=mesh,
        compiler_params=pltpu.CompilerParams(use_tc_tiling_on_sc=False),
        scratch_types=[
            pltpu.VMEM((_RING, 2, _CPE), jnp.int32),
            pltpu.VMEM((_RING, _CPE, dim_m), jnp.float32),
            pltpu.VMEM((_RING, _CPE // 2, 2 * dim_m), jnp.float32),
            pltpu.VMEM_SHARED((n_nodes, dim_m), jnp.float32),
            pltpu.SemaphoreType.DMA,
            pltpu.SemaphoreType.DMA,
            pltpu.SemaphoreType.DMA,
            pltpu.SemaphoreType.DMA,
            pltpu.SemaphoreType.DMA,
            pltpu.SemaphoreType.DMA,
            pltpu.SemaphoreType.DMA,
        ],
    )
    partials = sc(xm, em, comb).reshape(_NC, n_nodes, dim_m)

    out = pl.pallas_call(
        _out_body,
        grid=(n_nodes // blk_n,),
        in_specs=[pl.BlockSpec((blk_n, dim_in), lambda i: (i, 0)),
                  pl.BlockSpec((blk_n, dim_m), lambda i: (i, 0)),
                  pl.BlockSpec((blk_n, dim_m), lambda i: (i, 0)),
                  pl.BlockSpec((dim_in, dim_out), lambda i: (0, 0)),
                  pl.BlockSpec((dim_m, dim_out), lambda i: (0, 0)),
                  pl.BlockSpec((1, dim_out), lambda i: (0, 0))],
        out_specs=pl.BlockSpec((blk_n, dim_out), lambda i: (i, 0)),
        out_shape=jax.ShapeDtypeStruct((n_nodes, dim_out), jnp.float32),
    )(x, partials[0], partials[1],
      U_W[:, :dim_in].T, U_W[:, dim_in:].T, U_b.reshape(1, dim_out))
    return out


# speedup vs baseline: 4.0158x; 1.1387x over previous
import functools

import jax
import jax.numpy as jnp
from jax import lax
from jax.experimental import pallas as pl
from jax.experimental.pallas import tpu as pltpu
from jax.experimental.pallas import tpu_sc as plsc

_NC = 2
_NS = 16
_LANES = 16
_CPE = 80
_RING = 5
_NCHUNK = 125


def _em_body(efa_ref, efb_ref, w_ref, b_ref, o_ref):
    o_ref[:, 0:64] = jnp.dot(efa_ref[...], w_ref[...],
                             preferred_element_type=jnp.float32) + b_ref[...]
    o_ref[:, 64:128] = jnp.dot(efb_ref[...], w_ref[...],
                               preferred_element_type=jnp.float32) + b_ref[...]


def _xm_body(x2_ref, w_ref, o_ref):
    o_ref[...] = jnp.dot(x2_ref[...], w_ref[...],
                         preferred_element_type=jnp.float32)


def _out_body(x_ref, p0_ref, p1_ref, wx_ref, wa_ref, b_ref, o_ref):
    agg = p0_ref[...] + p1_ref[...]
    o_ref[...] = (jnp.dot(x_ref[...], wx_ref[...],
                          preferred_element_type=jnp.float32)
                  + jnp.dot(agg, wa_ref[...],
                            preferred_element_type=jnp.float32)
                  + b_ref[...])


def _sc_body(xm_hbm, em_hbm, src_hbm, dst_hbm, out_hbm,
             idx, gath, emb, agg_sh,
             semg0, semg1, semg2, semg3, semg4,
             semi0, semi1, semi2, semi3, semi4, sems0, sems1,
             *, n_nodes, dim_m):
    cid = lax.axis_index("c")
    sid = lax.axis_index("s")
    wid = sid * _NC + cid
    nodes_per_tile = n_nodes // _NS
    vregs = dim_m // _LANES
    half = _CPE // 2
    semg = [semg0, semg1, semg2, semg3, semg4]
    semi = [semi0, semi1, semi2, semi3, semi4]
    sems = [sems0, sems1]
    g0 = wid * _NCHUNK

    def gather_fire(c, b):
        pltpu.async_copy(xm_hbm.at[idx.at[b % _RING, 0]],
                         gath.at[b % _RING], semg[b % _RING])
        pltpu.async_copy(em_hbm.at[pl.ds((g0 + c) * half, half)],
                         emb.at[b % _RING], semg[b % _RING])

    def gather_wait(b):
        pltpu.make_async_copy(xm_hbm.at[pl.ds(0, _CPE)],
                              gath.at[b % _RING], semg[b % _RING]).wait()
        pltpu.make_async_copy(em_hbm.at[pl.ds(0, half)],
                              emb.at[b % _RING], semg[b % _RING]).wait()

    def idx_fire(c, b):
        pltpu.async_copy(src_hbm.at[pl.ds((g0 + c) * _CPE, _CPE)],
                         idx.at[b % _RING, 0], semi[b % _RING])
        pltpu.async_copy(dst_hbm.at[pl.ds((g0 + c) * _CPE, _CPE)],
                         idx.at[b % _RING, 1], semi[b % _RING])

    def idx_wait(b):
        for j in range(2):
            pltpu.make_async_copy(src_hbm.at[pl.ds(0, _CPE)],
                                  idx.at[b % _RING, j],
                                  semi[b % _RING]).wait()

    def scatter_fire(b):
        return pltpu.async_copy(gath.at[b % _RING],
                                agg_sh.at[idx.at[b % _RING, 1]],
                                sems[b % 2], add=True)

    def zrow(r, c):
        for cc in range(vregs):
            gath[0, r, pl.ds(cc * _LANES, _LANES)] = jnp.zeros((_LANES,),
                                                               jnp.float32)
        return c
    lax.fori_loop(0, _CPE, zrow, 0)
    node_base = sid * nodes_per_tile
    done = 0
    while done < nodes_per_tile:
        n = min(_CPE, nodes_per_tile - done)
        pltpu.sync_copy(gath.at[0, pl.ds(0, n)],
                        agg_sh.at[pl.ds(node_base + done, n)])
        done += n
    plsc.subcore_barrier()

    for k in range(3):
        idx_fire(k, k)
    for k in range(2):
        idx_wait(k)
        gather_fire(k, k)

    def outer(o, carry):
        descs = [None] * _RING
        for b in range(_RING):
            i = o * _RING + b
            gather_wait(b)

            def comp(j, cc_, b=b):
                for h in range(2):
                    for cc in range(vregs):
                        s = pl.ds(cc * _LANES, _LANES)
                        se = pl.ds(h * dim_m + cc * _LANES, _LANES)
                        v = gath[b, h * half + j, s] + emb[b, j, se]
                        gath[b, h * half + j, s] = jnp.maximum(v, 0.0)
                return cc_
            lax.fori_loop(0, half, comp, 0)

            descs[b] = scatter_fire(b)
            if b >= 2:
                descs[b - 2].wait()

            @pl.when(i + 2 < _NCHUNK)
            def _prefetch(i=i, b=b):
                idx_wait(b + 2)
                gather_fire(i + 2, b + 2)

            @pl.when(i + 3 < _NCHUNK)
            def _previdx(i=i, b=b):
                idx_fire(i + 3, b + 3)
        descs[_RING - 2].wait()
        descs[_RING - 1].wait()
        return carry
    lax.fori_loop(0, _NCHUNK // _RING, outer, 0)

    plsc.subcore_barrier()
    pltpu.sync_copy(agg_sh.at[pl.ds(node_base, nodes_per_tile)],
                    out_hbm.at[cid * _NS + sid])


def kernel(x, edge_features, edge_idx, batch_idx, M_W, M_b, U_W, U_b):
    n_nodes, dim_in = x.shape
    n_edges, dim_e = edge_features.shape
    dim_m = M_W.shape[0]
    dim_out = U_W.shape[0]

    n_workers = _NC * _NS
    assert n_edges == n_workers * _NCHUNK * _CPE
    e_half = n_edges // 2
    half = _CPE // 2
    n_rows = e_half // half

    def chunk_flat(v):
        lo = v[:e_half].reshape(n_rows, half)
        hi = v[e_half:].reshape(n_rows, half)
        return jnp.concatenate([lo, hi], axis=1).reshape(n_edges)
    srcc = chunk_flat(edge_idx[0])
    dstc = chunk_flat(edge_idx[1])

    blk_e = 4000
    grid_e = e_half // blk_e
    em = pl.pallas_call(
        _em_body,
        grid=(grid_e,),
        in_specs=[pl.BlockSpec((blk_e, dim_e), lambda i: (i, 0)),
                  pl.BlockSpec((blk_e, dim_e), lambda i, g=grid_e: (i + g, 0)),
                  pl.BlockSpec((dim_e, dim_m), lambda i: (0, 0)),
                  pl.BlockSpec((1, dim_m), lambda i: (0, 0))],
        out_specs=pl.BlockSpec((blk_e, 2 * dim_m), lambda i: (i, 0)),
        out_shape=jax.ShapeDtypeStruct((e_half, 2 * dim_m), jnp.float32),
    )(edge_features, edge_features, M_W[:, dim_in:].T, M_b.reshape(1, dim_m))

    pair = 128 // dim_m
    w_x = M_W[:, :dim_in]
    w2x = jnp.zeros((pair * dim_in, pair * dim_m), jnp.float32)
    for s in range(pair):
        w2x = w2x.at[s * dim_in:(s + 1) * dim_in,
                     s * dim_m:(s + 1) * dim_m].set(w_x.T)
    blk_n = 2000
    x2 = x.reshape(n_nodes // pair, pair * dim_in)
    xm2 = pl.pallas_call(
        _xm_body,
        grid=(n_nodes // pair // (blk_n // pair),),
        in_specs=[pl.BlockSpec((blk_n // pair, pair * dim_in), lambda i: (i, 0)),
                  pl.BlockSpec((pair * dim_in, pair * dim_m), lambda i: (0, 0))],
        out_specs=pl.BlockSpec((blk_n // pair, pair * dim_m), lambda i: (i, 0)),
        out_shape=jax.ShapeDtypeStruct((n_nodes // pair, pair * dim_m),
                                       jnp.float32),
    )(x2, w2x)
    xm = xm2.reshape(n_nodes, dim_m)

    mesh = plsc.VectorSubcoreMesh(core_axis_name="c", subcore_axis_name="s",
                                  num_cores=_NC, num_subcores=_NS)
    sc = pl.kernel(
        functools.partial(_sc_body, n_nodes=n_nodes, dim_m=dim_m),
        out_type=jax.ShapeDtypeStruct((_NC * _NS, n_nodes // _NS, dim_m),
                                      jnp.float32),
        mesh=mesh,
        compiler_params=pltpu.CompilerParams(use_tc_tiling_on_sc=False),
        scratch_types=[
            pltpu.VMEM((_RING, 2, _CPE), jnp.int32),
            pltpu.VMEM((_RING, _CPE, dim_m), jnp.float32),
            pltpu.VMEM((_RING, _CPE // 2, 2 * dim_m), jnp.float32),
            pltpu.VMEM_SHARED((n_nodes, dim_m), jnp.float32),
        ] + [pltpu.SemaphoreType.DMA] * 12,
    )
    partials = sc(xm, em, srcc, dstc).reshape(_NC, n_nodes, dim_m)

    out = pl.pallas_call(
        _out_body,
        grid=(n_nodes // blk_n,),
        in_specs=[pl.BlockSpec((blk_n, dim_in), lambda i: (i, 0)),
                  pl.BlockSpec((blk_n, dim_m), lambda i: (i, 0)),
                  pl.BlockSpec((blk_n, dim_m), lambda i: (i, 0)),
                  pl.BlockSpec((dim_in, dim_out), lambda i: (0, 0)),
                  pl.BlockSpec((dim_m, dim_out), lambda i: (0, 0)),
                  pl.BlockSpec((1, dim_out), lambda i: (0, 0))],
        out_specs=pl.BlockSpec((blk_n, dim_out), lambda i: (i, 0)),
        out_shape=jax.ShapeDtypeStruct((n_nodes, dim_out), jnp.float32),
    )(x, partials[0], partials[1],
      U_W[:, :dim_in].T, U_W[:, dim_in:].T, U_b.reshape(1, dim_out))
    return out


# speedup vs baseline: 4.0191x; 1.0008x over previous
import functools

import jax
import jax.numpy as jnp
from jax import lax
from jax.experimental import pallas as pl
from jax.experimental.pallas import tpu as pltpu
from jax.experimental.pallas import tpu_sc as plsc

_NC = 2
_NS = 16
_LANES = 16
_CPE = 80
_RING = 5
_NCHUNK = 125


def _em_body(efa_ref, efb_ref, w_ref, b_ref, o_ref):
    blk = efa_ref.shape[0]
    lo = jnp.dot(efa_ref[...], w_ref[...],
                 preferred_element_type=jnp.float32) + b_ref[...]
    hi = jnp.dot(efb_ref[...], w_ref[...],
                 preferred_element_type=jnp.float32) + b_ref[...]
    acc = jnp.concatenate([lo, hi], axis=1)
    o_ref[...] = acc.reshape(blk // (_CPE // 2), _CPE // 2, 128)


def _xm_body(x2_ref, w_ref, o_ref):
    o_ref[...] = jnp.dot(x2_ref[...], w_ref[...],
                         preferred_element_type=jnp.float32)


def _out_body(x_ref, p0_ref, p1_ref, wx_ref, wa_ref, b_ref, o_ref):
    agg = p0_ref[...] + p1_ref[...]
    o_ref[...] = (jnp.dot(x_ref[...], wx_ref[...],
                          preferred_element_type=jnp.float32)
                  + jnp.dot(agg, wa_ref[...],
                            preferred_element_type=jnp.float32)
                  + b_ref[...])


def _sc_body(xm_hbm, em_hbm, src_hbm, dst_hbm, out_hbm,
             idx, gath, emb, agg_sh,
             semg0, semg1, semg2, semg3, semg4,
             semi0, semi1, semi2, semi3, semi4, sems0, sems1,
             *, n_nodes, dim_m):
    cid = lax.axis_index("c")
    sid = lax.axis_index("s")
    wid = sid * _NC + cid
    nodes_per_tile = n_nodes // _NS
    vregs = dim_m // _LANES
    half = _CPE // 2
    semg = [semg0, semg1, semg2, semg3, semg4]
    semi = [semi0, semi1, semi2, semi3, semi4]
    sems = [sems0, sems1]
    g0 = wid * _NCHUNK

    def gather_fire(c, b):
        pltpu.async_copy(xm_hbm.at[idx.at[b % _RING, 0]],
                         gath.at[b % _RING], semg[b % _RING])
        pltpu.async_copy(em_hbm.at[g0 + c],
                         emb.at[b % _RING], semg[b % _RING])

    def gather_wait(b):
        pltpu.make_async_copy(xm_hbm.at[pl.ds(0, _CPE)],
                              gath.at[b % _RING], semg[b % _RING]).wait()
        pltpu.make_async_copy(em_hbm.at[0],
                              emb.at[b % _RING], semg[b % _RING]).wait()

    def idx_fire(c, b):
        pltpu.async_copy(src_hbm.at[pl.ds((g0 + c) * _CPE, _CPE)],
                         idx.at[b % _RING, 0], semi[b % _RING])
        pltpu.async_copy(dst_hbm.at[pl.ds((g0 + c) * _CPE, _CPE)],
                         idx.at[b % _RING, 1], semi[b % _RING])

    def idx_wait(b):
        for j in range(2):
            pltpu.make_async_copy(src_hbm.at[pl.ds(0, _CPE)],
                                  idx.at[b % _RING, j],
                                  semi[b % _RING]).wait()

    def scatter_fire(b):
        return pltpu.async_copy(gath.at[b % _RING],
                                agg_sh.at[idx.at[b % _RING, 1]],
                                sems[b % 2], add=True)

    def zrow(r, c):
        for cc in range(vregs):
            gath[0, r, pl.ds(cc * _LANES, _LANES)] = jnp.zeros((_LANES,),
                                                               jnp.float32)
        return c
    lax.fori_loop(0, _CPE, zrow, 0)
    node_base = sid * nodes_per_tile
    done = 0
    while done < nodes_per_tile:
        n = min(_CPE, nodes_per_tile - done)
        pltpu.sync_copy(gath.at[0, pl.ds(0, n)],
                        agg_sh.at[pl.ds(node_base + done, n)])
        done += n
    plsc.subcore_barrier()

    for k in range(3):
        idx_fire(k, k)
    for k in range(2):
        idx_wait(k)
        gather_fire(k, k)

    def outer(o, carry):
        descs = [None] * _RING
        for b in range(_RING):
            i = o * _RING + b
            gather_wait(b)

            def comp(j, cc_, b=b):
                for h in range(2):
                    for cc in range(vregs):
                        s = pl.ds(cc * _LANES, _LANES)
                        se = pl.ds(h * dim_m + cc * _LANES, _LANES)
                        v = gath[b, h * half + j, s] + emb[b, j, se]
                        gath[b, h * half + j, s] = jnp.maximum(v, 0.0)
                return cc_
            lax.fori_loop(0, half, comp, 0)

            descs[b] = scatter_fire(b)
            if b >= 2:
                descs[b - 2].wait()

            @pl.when(i + 2 < _NCHUNK)
            def _prefetch(i=i, b=b):
                idx_wait(b + 2)
                gather_fire(i + 2, b + 2)

            @pl.when(i + 3 < _NCHUNK)
            def _previdx(i=i, b=b):
                idx_fire(i + 3, b + 3)
        descs[_RING - 2].wait()
        descs[_RING - 1].wait()
        return carry
    lax.fori_loop(0, _NCHUNK // _RING, outer, 0)

    plsc.subcore_barrier()
    pltpu.sync_copy(agg_sh.at[pl.ds(node_base, nodes_per_tile)],
                    out_hbm.at[cid * _NS + sid])


def kernel(x, edge_features, edge_idx, batch_idx, M_W, M_b, U_W, U_b):
    n_nodes, dim_in = x.shape
    n_edges, dim_e = edge_features.shape
    dim_m = M_W.shape[0]
    dim_out = U_W.shape[0]

    n_workers = _NC * _NS
    assert n_edges == n_workers * _NCHUNK * _CPE
    e_half = n_edges // 2
    half = _CPE // 2
    n_rows = e_half // half

    def chunk_flat(v):
        lo = v[:e_half].reshape(n_rows, half)
        hi = v[e_half:].reshape(n_rows, half)
        return jnp.concatenate([lo, hi], axis=1).reshape(n_edges)
    srcc = chunk_flat(edge_idx[0])
    dstc = chunk_flat(edge_idx[1])

    blk_e = 4000
    grid_e = e_half // blk_e
    em = pl.pallas_call(
        _em_body,
        grid=(grid_e,),
        in_specs=[pl.BlockSpec((blk_e, dim_e), lambda i: (i, 0)),
                  pl.BlockSpec((blk_e, dim_e), lambda i, g=grid_e: (i + g, 0)),
                  pl.BlockSpec((dim_e, dim_m), lambda i: (0, 0)),
                  pl.BlockSpec((1, dim_m), lambda i: (0, 0))],
        out_specs=pl.BlockSpec((blk_e // half, half, 2 * dim_m),
                               lambda i: (i, 0, 0)),
        out_shape=jax.ShapeDtypeStruct((e_half // half, half, 2 * dim_m),
                                       jnp.float32),
    )(edge_features, edge_features, M_W[:, dim_in:].T, M_b.reshape(1, dim_m))

    pair = 128 // dim_m
    w_x = M_W[:, :dim_in]
    w2x = jnp.zeros((pair * dim_in, pair * dim_m), jnp.float32)
    for s in range(pair):
        w2x = w2x.at[s * dim_in:(s + 1) * dim_in,
                     s * dim_m:(s + 1) * dim_m].set(w_x.T)
    blk_n = 2000
    x2 = x.reshape(n_nodes // pair, pair * dim_in)
    xm2 = pl.pallas_call(
        _xm_body,
        grid=(n_nodes // pair // (blk_n // pair),),
        in_specs=[pl.BlockSpec((blk_n // pair, pair * dim_in), lambda i: (i, 0)),
                  pl.BlockSpec((pair * dim_in, pair * dim_m), lambda i: (0, 0))],
        out_specs=pl.BlockSpec((blk_n // pair, pair * dim_m), lambda i: (i, 0)),
        out_shape=jax.ShapeDtypeStruct((n_nodes // pair, pair * dim_m),
                                       jnp.float32),
    )(x2, w2x)
    xm = xm2.reshape(n_nodes, dim_m)

    mesh = plsc.VectorSubcoreMesh(core_axis_name="c", subcore_axis_name="s",
                                  num_cores=_NC, num_subcores=_NS)
    sc = pl.kernel(
        functools.partial(_sc_body, n_nodes=n_nodes, dim_m=dim_m),
        out_type=jax.ShapeDtypeStruct((_NC * _NS, n_nodes // _NS, dim_m),
                                      jnp.float32),
        mesh=mesh,
        compiler_params=pltpu.CompilerParams(use_tc_tiling_on_sc=False),
        scratch_types=[
            pltpu.VMEM((_RING, 2, _CPE), jnp.int32),
            pltpu.VMEM((_RING, _CPE, dim_m), jnp.float32),
            pltpu.VMEM((_RING, _CPE // 2, 2 * dim_m), jnp.float32),
            pltpu.VMEM_SHARED((n_nodes, dim_m), jnp.float32),
        ] + [pltpu.SemaphoreType.DMA] * 12,
    )
    partials = sc(xm, em, srcc, dstc).reshape(_NC, n_nodes, dim_m)

    out = pl.pallas_call(
        _out_body,
        grid=(n_nodes // blk_n,),
        in_specs=[pl.BlockSpec((blk_n, dim_in), lambda i: (i, 0)),
                  pl.BlockSpec((blk_n, dim_m), lambda i: (i, 0)),
                  pl.BlockSpec((blk_n, dim_m), lambda i: (i, 0)),
                  pl.BlockSpec((dim_in, dim_out), lambda i: (0, 0)),
                  pl.BlockSpec((dim_m, dim_out), lambda i: (0, 0)),
                  pl.BlockSpec((1, dim_out), lambda i: (0, 0))],
        out_specs=pl.BlockSpec((blk_n, dim_out), lambda i: (i, 0)),
        out_shape=jax.ShapeDtypeStruct((n_nodes, dim_out), jnp.float32),
    )(x, partials[0], partials[1],
      U_W[:, :dim_in].T, U_W[:, dim_in:].T, U_b.reshape(1, dim_out))
    return out


# speedup vs baseline: 4.8283x; 1.2013x over previous
import functools

import jax
import jax.numpy as jnp
from jax import lax
from jax.experimental import pallas as pl
from jax.experimental.pallas import tpu as pltpu
from jax.experimental.pallas import tpu_sc as plsc

_NC = 2
_NS = 16
_LANES = 16
_CPE = 80
_RING = 5
_NCHUNK = 125


def _em_body(efa_ref, efb_ref, w_ref, b_ref, o_ref):
    dn = (((0,), (0,)), ((), ()))
    blk = efa_ref.shape[1]
    lo = lax.dot_general(efa_ref[...], w_ref[...], dn,
                         preferred_element_type=jnp.float32) + b_ref[...]
    hi = lax.dot_general(efb_ref[...], w_ref[...], dn,
                         preferred_element_type=jnp.float32) + b_ref[...]
    acc = jnp.concatenate([lo, hi], axis=1)
    o_ref[...] = acc.reshape(blk // (_CPE // 2), _CPE // 2, 128)


def _xm_body(x2_ref, w_ref, o_ref):
    o_ref[...] = jnp.dot(x2_ref[...], w_ref[...],
                         preferred_element_type=jnp.float32)


def _out_body(x_ref, p0_ref, p1_ref, wx_ref, wa_ref, b_ref, o_ref):
    agg = p0_ref[...] + p1_ref[...]
    o_ref[...] = (jnp.dot(x_ref[...], wx_ref[...],
                          preferred_element_type=jnp.float32)
                  + jnp.dot(agg, wa_ref[...],
                            preferred_element_type=jnp.float32)
                  + b_ref[...])


def _sc_body(xm_hbm, em_hbm, src_hbm, dst_hbm, out_hbm,
             idx, gath, emb, agg_sh,
             semg0, semg1, semg2, semg3, semg4,
             semi0, semi1, semi2, semi3, semi4, sems0, sems1,
             *, n_nodes, dim_m):
    cid = lax.axis_index("c")
    sid = lax.axis_index("s")
    wid = sid * _NC + cid
    nodes_per_tile = n_nodes // _NS
    vregs = dim_m // _LANES
    half = _CPE // 2
    semg = [semg0, semg1, semg2, semg3, semg4]
    semi = [semi0, semi1, semi2, semi3, semi4]
    sems = [sems0, sems1]
    g0 = wid * _NCHUNK

    def gather_fire(c, b):
        pltpu.async_copy(xm_hbm.at[idx.at[b % _RING, 0]],
                         gath.at[b % _RING], semg[b % _RING])
        pltpu.async_copy(em_hbm.at[g0 + c],
                         emb.at[b % _RING], semg[b % _RING])

    def gather_wait(b):
        pltpu.make_async_copy(xm_hbm.at[pl.ds(0, _CPE)],
                              gath.at[b % _RING], semg[b % _RING]).wait()
        pltpu.make_async_copy(em_hbm.at[0],
                              emb.at[b % _RING], semg[b % _RING]).wait()

    def idx_fire(c, b):
        pltpu.async_copy(src_hbm.at[pl.ds((g0 + c) * _CPE, _CPE)],
                         idx.at[b % _RING, 0], semi[b % _RING])
        pltpu.async_copy(dst_hbm.at[pl.ds((g0 + c) * _CPE, _CPE)],
                         idx.at[b % _RING, 1], semi[b % _RING])

    def idx_wait(b):
        for j in range(2):
            pltpu.make_async_copy(src_hbm.at[pl.ds(0, _CPE)],
                                  idx.at[b % _RING, j],
                                  semi[b % _RING]).wait()

    def scatter_fire(b):
        return pltpu.async_copy(gath.at[b % _RING],
                                agg_sh.at[idx.at[b % _RING, 1]],
                                sems[b % 2], add=True)

    def zrow(r, c):
        for cc in range(vregs):
            gath[0, r, pl.ds(cc * _LANES, _LANES)] = jnp.zeros((_LANES,),
                                                               jnp.float32)
        return c
    lax.fori_loop(0, _CPE, zrow, 0)
    node_base = sid * nodes_per_tile
    done = 0
    while done < nodes_per_tile:
        n = min(_CPE, nodes_per_tile - done)
        pltpu.sync_copy(gath.at[0, pl.ds(0, n)],
                        agg_sh.at[pl.ds(node_base + done, n)])
        done += n
    plsc.subcore_barrier()

    for k in range(3):
        idx_fire(k, k)
    for k in range(2):
        idx_wait(k)
        gather_fire(k, k)

    def outer(o, carry):
        descs = [None] * _RING
        for b in range(_RING):
            i = o * _RING + b
            gather_wait(b)

            def comp(j, cc_, b=b):
                for h in range(2):
                    for cc in range(vregs):
                        s = pl.ds(cc * _LANES, _LANES)
                        se = pl.ds(h * dim_m + cc * _LANES, _LANES)
                        v = gath[b, h * half + j, s] + emb[b, j, se]
                        gath[b, h * half + j, s] = jnp.maximum(v, 0.0)
                return cc_
            lax.fori_loop(0, half, comp, 0)

            descs[b] = scatter_fire(b)
            if b >= 2:
                descs[b - 2].wait()

            @pl.when(i + 2 < _NCHUNK)
            def _prefetch(i=i, b=b):
                idx_wait(b + 2)
                gather_fire(i + 2, b + 2)

            @pl.when(i + 3 < _NCHUNK)
            def _previdx(i=i, b=b):
                idx_fire(i + 3, b + 3)
        descs[_RING - 2].wait()
        descs[_RING - 1].wait()
        return carry
    lax.fori_loop(0, _NCHUNK // _RING, outer, 0)

    plsc.subcore_barrier()
    pltpu.sync_copy(agg_sh.at[pl.ds(node_base, nodes_per_tile)],
                    out_hbm.at[cid * _NS + sid])


def kernel(x, edge_features, edge_idx, batch_idx, M_W, M_b, U_W, U_b):
    n_nodes, dim_in = x.shape
    n_edges, dim_e = edge_features.shape
    dim_m = M_W.shape[0]
    dim_out = U_W.shape[0]

    n_workers = _NC * _NS
    assert n_edges == n_workers * _NCHUNK * _CPE
    e_half = n_edges // 2
    half = _CPE // 2
    n_rows = e_half // half

    def chunk_flat(v):
        lo = v[:e_half].reshape(n_rows, half)
        hi = v[e_half:].reshape(n_rows, half)
        return jnp.concatenate([lo, hi], axis=1).reshape(n_edges)
    srcc = chunk_flat(edge_idx[0])
    dstc = chunk_flat(edge_idx[1])

    blk_e = 6400
    grid_e = e_half // blk_e
    em = pl.pallas_call(
        _em_body,
        grid=(grid_e,),
        in_specs=[pl.BlockSpec((dim_e, blk_e), lambda i: (0, i)),
                  pl.BlockSpec((dim_e, blk_e), lambda i, g=grid_e: (0, i + g)),
                  pl.BlockSpec((dim_e, dim_m), lambda i: (0, 0)),
                  pl.BlockSpec((1, dim_m), lambda i: (0, 0))],
        out_specs=pl.BlockSpec((blk_e // half, half, 2 * dim_m),
                               lambda i: (i, 0, 0)),
        out_shape=jax.ShapeDtypeStruct((e_half // half, half, 2 * dim_m),
                                       jnp.float32),
    )(edge_features.T, edge_features.T, M_W[:, dim_in:].T,
      M_b.reshape(1, dim_m))

    pair = 128 // dim_m
    w_x = M_W[:, :dim_in]
    w2x = jnp.zeros((pair * dim_in, pair * dim_m), jnp.float32)
    for s in range(pair):
        w2x = w2x.at[s * dim_in:(s + 1) * dim_in,
                     s * dim_m:(s + 1) * dim_m].set(w_x.T)
    blk_n = 2000
    x2 = x.reshape(n_nodes // pair, pair * dim_in)
    xm2 = pl.pallas_call(
        _xm_body,
        grid=(n_nodes // pair // (blk_n // pair),),
        in_specs=[pl.BlockSpec((blk_n // pair, pair * dim_in), lambda i: (i, 0)),
                  pl.BlockSpec((pair * dim_in, pair * dim_m), lambda i: (0, 0))],
        out_specs=pl.BlockSpec((blk_n // pair, pair * dim_m), lambda i: (i, 0)),
        out_shape=jax.ShapeDtypeStruct((n_nodes // pair, pair * dim_m),
                                       jnp.float32),
    )(x2, w2x)
    xm = xm2.reshape(n_nodes, dim_m)

    mesh = plsc.VectorSubcoreMesh(core_axis_name="c", subcore_axis_name="s",
                                  num_cores=_NC, num_subcores=_NS)
    sc = pl.kernel(
        functools.partial(_sc_body, n_nodes=n_nodes, dim_m=dim_m),
        out_type=jax.ShapeDtypeStruct((_NC * _NS, n_nodes // _NS, dim_m),
                                      jnp.float32),
        mesh=mesh,
        compiler_params=pltpu.CompilerParams(use_tc_tiling_on_sc=False),
        scratch_types=[
            pltpu.VMEM((_RING, 2, _CPE), jnp.int32),
            pltpu.VMEM((_RING, _CPE, dim_m), jnp.float32),
            pltpu.VMEM((_RING, _CPE // 2, 2 * dim_m), jnp.float32),
            pltpu.VMEM_SHARED((n_nodes, dim_m), jnp.float32),
        ] + [pltpu.SemaphoreType.DMA] * 12,
    )
    partials = sc(xm, em, srcc, dstc).reshape(_NC, n_nodes, dim_m)

    out = pl.pallas_call(
        _out_body,
        grid=(n_nodes // blk_n,),
        in_specs=[pl.BlockSpec((blk_n, dim_in), lambda i: (i, 0)),
                  pl.BlockSpec((blk_n, dim_m), lambda i: (i, 0)),
                  pl.BlockSpec((blk_n, dim_m), lambda i: (i, 0)),
                  pl.BlockSpec((dim_in, dim_out), lambda i: (0, 0)),
                  pl.BlockSpec((dim_m, dim_out), lambda i: (0, 0)),
                  pl.BlockSpec((1, dim_out), lambda i: (0, 0))],
        out_specs=pl.BlockSpec((blk_n, dim_out), lambda i: (i, 0)),
        out_shape=jax.ShapeDtypeStruct((n_nodes, dim_out), jnp.float32),
    )(x, partials[0], partials[1],
      U_W[:, :dim_in].T, U_W[:, dim_in:].T, U_b.reshape(1, dim_out))
    return out
